# baseline (speedup 1.0000x reference)
import functools

import jax
import jax.numpy as jnp
from jax.experimental import pallas as pl
from jax.experimental.pallas import tpu as pltpu

f32 = jnp.float32


def _mm_body(x_ref, w_ref, b_ref, o_ref):
    o_ref[...] = (
        jnp.dot(x_ref[...], w_ref[...], preferred_element_type=jnp.float32)
        + b_ref[...]
    )


def _mm(x, w, b, bm=512):
    M, K = x.shape
    Nc = w.shape[1]
    Mp = ((M + bm - 1) // bm) * bm
    if Mp != M:
        x = jnp.pad(x, ((0, Mp - M), (0, 0)))
    out = pl.pallas_call(
        _mm_body,
        grid=(Mp // bm,),
        in_specs=[
            pl.BlockSpec((bm, K), lambda i: (i, 0)),
            pl.BlockSpec((K, Nc), lambda i: (0, 0)),
            pl.BlockSpec((1, Nc), lambda i: (0, 0)),
        ],
        out_specs=pl.BlockSpec((bm, Nc), lambda i: (i, 0)),
        out_shape=jax.ShapeDtypeStruct((Mp, Nc), f32),
    )(x, w, b.reshape(1, Nc))
    return out[:M]


def _bnrelu_mm_body(x_ref, ef_ref, a_ref, d_ref, w_ref, u_ref, o_ref):
    t = jnp.maximum(x_ref[...] * a_ref[...] + d_ref[...], 0.0)
    o_ref[...] = (
        jnp.dot(t, w_ref[...], preferred_element_type=jnp.float32)
        + ef_ref[...] * u_ref[...]
    )


def _bnrelu_mm(x, ef, a, d, w, u, bm=1000):
    M, K = x.shape
    Nc = w.shape[1]
    assert M % bm == 0
    return pl.pallas_call(
        _bnrelu_mm_body,
        grid=(M // bm,),
        in_specs=[
            pl.BlockSpec((bm, K), lambda i: (i, 0)),
            pl.BlockSpec((bm, 1), lambda i: (i, 0)),
            pl.BlockSpec((1, K), lambda i: (0, 0)),
            pl.BlockSpec((1, K), lambda i: (0, 0)),
            pl.BlockSpec((K, Nc), lambda i: (0, 0)),
            pl.BlockSpec((1, Nc), lambda i: (0, 0)),
        ],
        out_specs=pl.BlockSpec((bm, Nc), lambda i: (i, 0)),
        out_shape=jax.ShapeDtypeStruct((M, Nc), f32),
    )(x, ef.reshape(M, 1), a.reshape(1, K), d.reshape(1, K), w,
      u.reshape(1, Nc))


def kernel(node_feat, pos_enc, edge_feat, snorm_n, targets, edge_index, params):
    N = node_feat.shape[0]
    D = params['emb_h_W'].shape[1]
    src = edge_index[0]
    dst = edge_index[1]
    ef = edge_feat[:, 0]

    h = _mm(node_feat, params['emb_h_W'], params['emb_h_b'])
    p = _mm(pos_enc, params['emb_p_W'], params['emb_p_b'])
    w_e = params['emb_e_W'][0]
    b_e = params['emb_e_b']
    t_sum = None
    n_layers = len(params['layers'])
    Z = jnp.zeros((D, D), f32)

    for li, lp in enumerate(params['layers']):
        last = li == n_layers - 1
        X = jnp.concatenate([h, p], axis=1)
        Wall = jnp.concatenate([
            lp['A1_W'],
            lp['A2_W'],
            jnp.concatenate([lp['B1_W'], Z], axis=0),
            jnp.concatenate([lp['B2_W'], Z], axis=0),
            jnp.concatenate([Z, lp['C1_W']], axis=0),
            jnp.concatenate([Z, lp['C2_W']], axis=0),
        ], axis=1)
        u = w_e @ lp['B3_W']
        cvec = b_e @ lp['B3_W'] + lp['B3_b']
        ball = jnp.concatenate([
            lp['A1_b'], lp['A2_b'], lp['B1_b'] + cvec, lp['B2_b'],
            lp['C1_b'], lp['C2_b'],
        ])
        Y = _mm(X, Wall, ball)
        A1h = Y[:, 0 * D:1 * D]
        A2h = Y[:, 1 * D:2 * D]
        B1h = Y[:, 2 * D:3 * D]
        B2h = Y[:, 3 * D:4 * D]
        C1p = Y[:, 4 * D:5 * D]
        C2p = Y[:, 5 * D:6 * D]

        if t_sum is None:
            hat = B1h[src] + B2h[dst] + ef[:, None] * u[None, :]
        else:
            b3c = _bnrelu_mm(t_sum[0], ef, t_sum[1], t_sum[2], lp['B3_W'], u)
            hat = B1h[src] + B2h[dst] + b3c
        sigma = jax.nn.sigmoid(hat)
        sum_sigma = jax.ops.segment_sum(sigma, dst, num_segments=N)
        eta = sigma / (sum_sigma[dst] + 1e-6)
        hmsg = jax.ops.segment_sum(eta * A2h[src], dst, num_segments=N)
        pmsg = jax.ops.segment_sum(eta * C2p[src], dst, num_segments=N)

        h_new = (A1h + hmsg) * snorm_n
        m = jnp.mean(h_new, axis=0)
        v = jnp.var(h_new, axis=0)
        h_new = (h_new - m) / jnp.sqrt(v + 1e-5) * lp['bn_h_g'] + lp['bn_h_b']
        h = h + jnp.maximum(h_new, 0.0)
        p = p + jnp.tanh(C1p + pmsg)

        if not last:
            me = jnp.mean(hat, axis=0)
            ve = jnp.var(hat, axis=0)
            a = lp['bn_e_g'] / jnp.sqrt(ve + 1e-5)
            dshift = lp['bn_e_b'] - me * a
            t_sum = (hat, a, dshift)

    pp = p @ params['p_out_W'] + params['p_out_b']
    mu = jnp.mean(pp, axis=0)
    ppc = pp - mu
    denom = jnp.sqrt(jnp.sum(ppc * ppc, axis=0))
    p0 = ppc[0] / denom
    hp0 = jnp.concatenate([h[0], p0]) @ params['Whp_W'] + params['Whp_b']
    y = hp0.reshape(1, -1)
    n_mlp = len(params['mlp'])
    for i, (W, b) in enumerate(params['mlp']):
        y = y @ W + b
        if i < n_mlp - 1:
            y = jax.nn.relu(y)
    return (y, targets)
